# baseline (speedup 1.0000x reference)
import functools
import jax
import jax.numpy as jnp
from jax import lax
from jax.experimental import pallas as pl
from jax.experimental.pallas import tpu as pltpu
from jax.experimental.pallas import tpu_sc as plsc

N = 10000
E = 320000
NC, NS, L = 2, 16, 16
NW = NC * NS
EPW = E // NW
CK = 80
NCH = EPW // CK
NPAD = 10240


def _zero_vmem(buf, n):
    zeros = jnp.zeros((L,), jnp.float32)
    for i in range(n // L):
        buf[pl.ds(i * L, L)] = zeros


_MESH = plsc.VectorSubcoreMesh(core_axis_name="c", subcore_axis_name="s")


@functools.partial(
    pl.kernel,
    out_type=jax.ShapeDtypeStruct((NC, 2, NPAD), jnp.float32),
    mesh=_MESH,
    scratch_types=[
        pltpu.VMEM((NCH, CK), jnp.int32),
        pltpu.VMEM((NCH, CK), jnp.int32),
        pltpu.VMEM((CK,), jnp.float32),
        pltpu.VMEM((640,), jnp.float32),
        pltpu.VMEM_SHARED((NPAD,), jnp.float32),
        pltpu.VMEM_SHARED((NPAD,), jnp.float32),
        pltpu.SemaphoreType.DMA,
    ],
)
def _deg_kernel(g_hbm, out_hbm, sbuf, dbuf, ones, zbuf, acc_s, acc_d, dsem):
    c = lax.axis_index("c")
    s = lax.axis_index("s")
    one = jnp.ones((L,), jnp.float32)
    for i in range(CK // L):
        ones[pl.ds(i * L, L)] = one
    _zero_vmem(zbuf, 640)
    pltpu.sync_copy(zbuf, acc_s.at[pl.ds(s * 640, 640)])
    pltpu.sync_copy(zbuf, acc_d.at[pl.ds(s * 640, 640)])
    plsc.subcore_barrier()
    wid = s * NC + c
    pltpu.sync_copy(g_hbm.at[0, wid], sbuf)
    pltpu.sync_copy(g_hbm.at[1, wid], dbuf)

    def body(j, _):
        pltpu.async_copy(ones, acc_s.at[sbuf.at[j]], dsem, add=True)
        pltpu.async_copy(ones, acc_d.at[dbuf.at[j]], dsem, add=True)

        @pl.when(j >= 4)
        def _():
            pltpu.make_async_copy(ones, acc_s.at[sbuf.at[j - 4]], dsem).wait()
            pltpu.make_async_copy(ones, acc_d.at[dbuf.at[j - 4]], dsem).wait()

        return _

    lax.fori_loop(0, NCH, body, None)

    def drain(j, _):
        pltpu.make_async_copy(ones, acc_s.at[sbuf.at[j]], dsem).wait()
        pltpu.make_async_copy(ones, acc_d.at[dbuf.at[j]], dsem).wait()
        return _

    lax.fori_loop(NCH - 4, NCH, drain, None)
    plsc.subcore_barrier()

    @pl.when(s == 0)
    def _():
        pltpu.sync_copy(acc_s, out_hbm.at[c, 0])
        pltpu.sync_copy(acc_d, out_hbm.at[c, 1])


@functools.partial(
    pl.kernel,
    out_type=jax.ShapeDtypeStruct((NC, NPAD, 64), jnp.float32),
    mesh=_MESH,
    scratch_types=[
        pltpu.VMEM((NCH, CK), jnp.int32),
        pltpu.VMEM((NCH, CK), jnp.int32),
        pltpu.VMEM((CK, 64), jnp.float32),
        pltpu.VMEM((CK, 64), jnp.float32),
        pltpu.VMEM((64, 64), jnp.float32),
        pltpu.VMEM_SHARED((NPAD, 64), jnp.float32),
        pltpu.SemaphoreType.DMA,
        pltpu.SemaphoreType.DMA,
        pltpu.SemaphoreType.DMA,
        pltpu.SemaphoreType.DMA,
    ],
    compiler_params=pltpu.CompilerParams(use_tc_tiling_on_sc=False),
)
def _segsum_kernel(tbl_hbm, g_hbm, out_hbm, sbuf, dbuf, rb0, rb1, zrows, acc,
                   sem0, sem1, ssem0, ssem1):
    c = lax.axis_index("c")
    s = lax.axis_index("s")
    zeros = jnp.zeros((L,), jnp.float32)

    def zrow_body(i, _):
        for q in range(4):
            zrows[i, pl.ds(q * L, L)] = zeros
        return _

    lax.fori_loop(0, 64, zrow_body, None)

    def zacc_body(k, _):
        pltpu.sync_copy(zrows, acc.at[pl.ds(s * 640 + k * 64, 64)])
        return _

    lax.fori_loop(0, 10, zacc_body, None)
    plsc.subcore_barrier()

    wid = s * NC + c
    pltpu.sync_copy(g_hbm.at[0, wid], sbuf)
    pltpu.sync_copy(g_hbm.at[1, wid], dbuf)

    rbs = (rb0, rb1)
    sems = (sem0, sem1)
    ssems = (ssem0, ssem1)

    def wait_scatter(b, j):
        pltpu.make_async_copy(rbs[b], acc.at[dbuf.at[j]], ssems[b]).wait()

    pltpu.async_copy(tbl_hbm.at[sbuf.at[0]], rb0, sem0)

    def body(jj, _):
        for b in range(2):
            j = jj * 2 + b

            @pl.when(j >= 1)
            def _():
                wait_scatter(1 - b, j - 1)

            pltpu.async_copy(tbl_hbm.at[sbuf.at[j + 1]], rbs[1 - b], sems[1 - b])
            pltpu.make_async_copy(tbl_hbm.at[sbuf.at[j]], rbs[b], sems[b]).wait()
            pltpu.async_copy(rbs[b], acc.at[dbuf.at[j]], ssems[b], add=True)
        return _

    lax.fori_loop(0, (NCH - 1) // 2, body, None)
    j_last = NCH - 1
    wait_scatter(1, j_last - 1)
    pltpu.make_async_copy(tbl_hbm.at[sbuf.at[j_last]], rb0, sem0).wait()
    pltpu.async_copy(rb0, acc.at[dbuf.at[j_last]], ssem0, add=True)
    wait_scatter(0, j_last)
    plsc.subcore_barrier()
    pltpu.sync_copy(acc.at[pl.ds(s * 640, 640)], out_hbm.at[c, pl.ds(s * 640, 640)])


@functools.partial(
    pl.kernel,
    out_type=jax.ShapeDtypeStruct((E,), jnp.float32),
    mesh=_MESH,
    scratch_types=[
        pltpu.VMEM((NCH, CK), jnp.int32),
        pltpu.VMEM((NCH, CK), jnp.int32),
        pltpu.VMEM((CK, 128), jnp.float32),
        pltpu.VMEM((CK, 128), jnp.float32),
        pltpu.VMEM((CK, 128), jnp.float32),
        pltpu.VMEM((CK, 128), jnp.float32),
        pltpu.VMEM((CK,), jnp.float32),
        pltpu.VMEM((CK,), jnp.float32),
        pltpu.SemaphoreType.DMA,
        pltpu.SemaphoreType.DMA,
        pltpu.SemaphoreType.DMA,
        pltpu.SemaphoreType.DMA,
    ],
    compiler_params=pltpu.CompilerParams(
        use_tc_tiling_on_sc=False, needs_layout_passes=False
    ),
)
def _score_kernel(h_hbm, g_hbm, out_hbm, sbuf, dbuf, rs0, rd0, rs1, rd1, obuf0,
                  obuf1, sem0, sem1, osem0, osem1):
    c = lax.axis_index("c")
    s = lax.axis_index("s")
    wid = s * NC + c
    pltpu.sync_copy(g_hbm.at[0, wid], sbuf)
    pltpu.sync_copy(g_hbm.at[1, wid], dbuf)

    rss = (rs0, rs1)
    rds = (rd0, rd1)
    sems = (sem0, sem1)
    obufs = (obuf0, obuf1)
    osems = (osem0, osem1)

    def fire(j, b):
        pltpu.async_copy(h_hbm.at[sbuf.at[j]], rss[b], sems[b])
        pltpu.async_copy(h_hbm.at[dbuf.at[j]], rds[b], sems[b])

    def wait(j, b):
        pltpu.make_async_copy(h_hbm.at[sbuf.at[j]], rss[b], sems[b]).wait()
        pltpu.make_async_copy(h_hbm.at[dbuf.at[j]], rds[b], sems[b]).wait()

    def process(j, b):
        rs, rd = rss[b], rds[b]
        obuf = obufs[b]

        @pl.when(j >= 2)
        def _():
            pltpu.make_async_copy(
                obuf, out_hbm.at[pl.ds(wid * EPW + (j - 2) * CK, CK)], osems[b]
            ).wait()

        lanes = lax.iota(jnp.int32, L)

        def grp_body(gi, _):
            res = jnp.zeros((L,), jnp.float32)
            for t in range(L):
                e = gi * L + t
                p = [rs[e, pl.ds(q * L, L)] * rd[e, pl.ds(q * L, L)]
                     for q in range(8)]
                s1 = [p[0] + p[1], p[2] + p[3], p[4] + p[5], p[6] + p[7]]
                acc = (s1[0] + s1[1]) + (s1[2] + s1[3])
                res = jnp.where(lanes == t, jnp.sum(acc), res)
            obuf[pl.ds(gi * L, L)] = res
            return _

        lax.fori_loop(0, CK // L, grp_body, None)
        pltpu.async_copy(obuf, out_hbm.at[pl.ds(wid * EPW + j * CK, CK)], osems[b])

    fire(0, 0)

    def body(jj, _):
        for b in range(2):
            j = jj * 2 + b
            wait(j, b)
            fire(j + 1, 1 - b)
            process(j, b)
        return _

    lax.fori_loop(0, (NCH - 1) // 2, body, None)
    j_last = NCH - 1
    wait(j_last, 0)
    process(j_last, 0)
    pltpu.make_async_copy(
        obuf0, out_hbm.at[pl.ds(wid * EPW + j_last * CK, CK)], osem0).wait()
    pltpu.make_async_copy(
        obuf1, out_hbm.at[pl.ds(wid * EPW + (j_last - 1) * CK, CK)], osem1).wait()


NSEG = 313
ACC_R = 320
SCK = 2560
NCHS = E // SCK
NEG = -1e30
CAPB = 128
CAP = CAPB * 128


@functools.partial(
    pl.kernel,
    out_type=(
        jax.ShapeDtypeStruct((2, NW, CAP), jnp.int32),
        jax.ShapeDtypeStruct((NW, L), jnp.int32),
    ),
    mesh=_MESH,
    scratch_types=[
        pltpu.VMEM((SCK,), jnp.int32),
        pltpu.VMEM((SCK,), jnp.int32),
        pltpu.VMEM((SCK,), jnp.int32),
        pltpu.VMEM((SCK,), jnp.int32),
        pltpu.VMEM((256,), jnp.int32),
        pltpu.VMEM((256,), jnp.int32),
        pltpu.VMEM((L,), jnp.int32),
        pltpu.VMEM((128,), jnp.int32),
        pltpu.VMEM((128,), jnp.int32),
        pltpu.SemaphoreType.DMA,
        pltpu.SemaphoreType.DMA,
        pltpu.SemaphoreType.DMA,
    ],
    compiler_params=pltpu.CompilerParams(
        use_tc_tiling_on_sc=False, needs_layout_passes=False
    ),
)
def _segmax_kernel(g_hbm, lists_hbm, cnt_hbm, sb0, db0, sb1,
                   db1, psrc, pdst, cbuf, flsrc, fldst, sem0, sem1, lsem):
    c = lax.axis_index("c")
    s = lax.axis_index("s")
    wid = s * NC + c
    lo = wid * NSEG

    sbs = (sb0, sb1)
    dbs = (db0, db1)
    sems = (sem0, sem1)

    def fire(j, b):
        pltpu.async_copy(g_hbm.at[0, j], sbs[b], sems[b])
        pltpu.async_copy(g_hbm.at[1, j], dbs[b], sems[b])

    def wait(j, b):
        pltpu.make_async_copy(g_hbm.at[0, j], sbs[b], sems[b]).wait()
        pltpu.make_async_copy(g_hbm.at[1, j], dbs[b], sems[b]).wait()

    def wait_listwrite(nf_prev):
        @pl.when((nf_prev >= 0) & (nf_prev < CAPB))
        def _():
            pltpu.make_async_copy(
                flsrc, lists_hbm.at[0, wid, pl.ds(nf_prev * 128, 128)], lsem
            ).wait()
            pltpu.make_async_copy(
                fldst, lists_hbm.at[1, wid, pl.ds(nf_prev * 128, 128)], lsem
            ).wait()

    def flush(st):
        np, nf = st
        wait_listwrite(nf - 1)
        for i in range(8):
            flsrc[pl.ds(i * L, L)] = psrc[pl.ds(i * L, L)]
            fldst[pl.ds(i * L, L)] = pdst[pl.ds(i * L, L)]

        @pl.when(nf < CAPB)
        def _():
            pltpu.async_copy(flsrc, lists_hbm.at[0, wid, pl.ds(nf * 128, 128)],
                             lsem)
            pltpu.async_copy(fldst, lists_hbm.at[1, wid, pl.ds(nf * 128, 128)],
                             lsem)

        for i in range(8):
            psrc[pl.ds(i * L, L)] = psrc[pl.ds(128 + i * L, L)]
            pdst[pl.ds(i * L, L)] = pdst[pl.ds(128 + i * L, L)]
        return (np - 128, nf + 1)

    def maybe_flush(st):
        return lax.cond(st[0] >= 128, flush, lambda x: x, st)

    def process(b, st):
        sb, db = sbs[b], dbs[b]

        def batch(k, st):
            np, nf = st
            for i8 in range(8):
                base = (k * 8 + i8) * L
                sV = sb[pl.ds(base, L)]
                dV = db[pl.ds(base, L)]
                dl = dV - lo
                m = (dl >= 0) & (dl < NSEG)
                cnt = plsc.all_reduce_population_count(m)[0]
                plsc.store_compressed(psrc.at[pl.ds(np, L)], sV, mask=m)
                plsc.store_compressed(pdst.at[pl.ds(np, L)], dl, mask=m)
                np = np + cnt
            return maybe_flush((np, nf))

        return lax.fori_loop(0, SCK // 128, batch, st)

    fire(0, 0)

    def chunk_body(jj, st):
        for b in range(2):
            j = jj * 2 + b
            wait(j, b)
            fire(j + 1, 1 - b)
            st = process(b, st)
        return st

    st = lax.fori_loop(0, (NCHS - 1) // 2, chunk_body, (0, 0))
    j_last = NCHS - 1
    wait(j_last, 0)
    st = process(0, st)
    st = maybe_flush(st)

    zero16 = jnp.zeros((L,), jnp.int32)
    trash16 = jnp.full((L,), NSEG, jnp.int32)
    full = jnp.ones((L,), jnp.bool_)

    def pad16(st):
        np, nf = st
        plsc.store_compressed(psrc.at[pl.ds(np, L)], zero16, mask=full)
        plsc.store_compressed(pdst.at[pl.ds(np, L)], trash16, mask=full)
        return (np + L, nf)

    def pad_body(_, st):
        return lax.cond(st[0] < 128, pad16, lambda x: x, st)

    st = lax.fori_loop(0, 8, pad_body, st)
    st = flush(st)
    nf = st[1]
    wait_listwrite(nf - 1)

    for i in range(8):
        psrc[pl.ds(i * L, L)] = zero16
        pdst[pl.ds(i * L, L)] = trash16

    def padblk(nf):
        @pl.when(nf < CAPB)
        def _():
            pltpu.sync_copy(psrc.at[pl.ds(0, 128)],
                            lists_hbm.at[0, wid, pl.ds(nf * 128, 128)])
            pltpu.sync_copy(pdst.at[pl.ds(0, 128)],
                            lists_hbm.at[1, wid, pl.ds(nf * 128, 128)])
        return nf + 1

    def padblk_body(_, nf):
        return lax.cond(nf % 8 != 0, padblk, lambda x: x, nf)

    nf = lax.fori_loop(0, 7, padblk_body, nf)
    cbuf[pl.ds(0, L)] = jnp.full((L,), 0, jnp.int32) + (nf // 8)
    pltpu.sync_copy(cbuf, cnt_hbm.at[wid])


@functools.partial(
    pl.kernel,
    out_type=jax.ShapeDtypeStruct((NW, ACC_R, 64), jnp.float32),
    mesh=_MESH,
    scratch_types=[
        pltpu.VMEM((1024,), jnp.int32),
        pltpu.VMEM((1024,), jnp.int32),
        pltpu.VMEM((128, 64), jnp.float32),
        pltpu.VMEM((128, 64), jnp.float32),
        pltpu.VMEM((ACC_R, 64), jnp.float32),
        pltpu.VMEM((L,), jnp.int32),
        pltpu.SemaphoreType.DMA,
        pltpu.SemaphoreType.DMA,
    ],
    compiler_params=pltpu.CompilerParams(
        use_tc_tiling_on_sc=False, needs_layout_passes=False
    ),
)
def _segmax_apply(tbl_hbm, lists_hbm, cnt_hbm, out_hbm, lsrc, ldst, rf0, rf1,
                  accr, cbuf, sem0, sem1):
    c = lax.axis_index("c")
    s = lax.axis_index("s")
    wid = s * NC + c

    neg = jnp.full((L,), NEG, jnp.float32)

    def init_body(rr, _):
        for q in range(4):
            accr[rr, pl.ds(q * L, L)] = neg
        return _

    lax.fori_loop(0, ACC_R, init_body, None)

    pltpu.sync_copy(cnt_hbm.at[wid], cbuf)
    nch = cbuf[pl.ds(0, L)][0]

    rfs = (rf0, rf1)
    sems = (sem0, sem1)

    def chunk_body(cc, _):
        pltpu.sync_copy(lists_hbm.at[0, wid, pl.ds(cc * 1024, 1024)], lsrc)
        pltpu.sync_copy(lists_hbm.at[1, wid, pl.ds(cc * 1024, 1024)], ldst)
        pltpu.async_copy(tbl_hbm.at[lsrc.at[pl.ds(0, 128)]], rf0, sem0)
        for b in range(8):
            pb = b % 2
            pltpu.make_async_copy(
                tbl_hbm.at[lsrc.at[pl.ds(b * 128, 128)]], rfs[pb], sems[pb]
            ).wait()
            if b < 7:
                pltpu.async_copy(
                    tbl_hbm.at[lsrc.at[pl.ds((b + 1) * 128, 128)]],
                    rfs[1 - pb], sems[1 - pb])
            rf = rfs[pb]

            def fold(gg, _):
                dv = ldst[pl.ds(b * 128 + gg * L, L)]
                for t in range(L):
                    e = gg * L + t
                    dloc = dv[t]
                    cur = [accr[dloc, pl.ds(q * L, L)] for q in range(4)]
                    new = [rf[e, pl.ds(q * L, L)] for q in range(4)]
                    for q in range(4):
                        accr[dloc, pl.ds(q * L, L)] = jnp.maximum(cur[q], new[q])
                return _

            lax.fori_loop(0, 128 // L, fold, None)
        return _

    lax.fori_loop(0, nch, chunk_body, None)
    pltpu.sync_copy(accr, out_hbm.at[wid])


def _mm_body(x_ref, w_ref, o_ref):
    o_ref[...] = jnp.dot(x_ref[...], w_ref[...],
                         preferred_element_type=jnp.float32)


def _mm(x, w):
    M, K = x.shape
    F = w.shape[1]
    return pl.pallas_call(
        _mm_body,
        grid=(10,),
        in_specs=[
            pl.BlockSpec((M // 10, K), lambda i: (i, 0)),
            pl.BlockSpec((K, F), lambda i: (0, 0)),
        ],
        out_specs=pl.BlockSpec((M // 10, F), lambda i: (i, 0)),
        out_shape=jax.ShapeDtypeStruct((M, F), jnp.float32),
    )(x, w)


def kernel(g, x_n, x_e, Wn1, bn1, Wn2, bn2, Th1, Tb1, Ph1, Pb1, Th2, Tb2, Ph2, Pb2):
    src, dst = g[0], g[1]
    g4 = g.reshape(2, NW, NCH, CK)

    degp = _deg_kernel(g4)
    deg_out = degp[0, 0, :N] + degp[1, 0, :N]
    deg_in = degp[0, 1, :N] + degp[1, 1, :N]
    norm_s = jnp.where(deg_out > 0, deg_out, 1.0) ** -0.5
    norm_d = jnp.where(deg_in > 0, deg_in, 1.0) ** -0.5

    def gconv(x, W, b):
        y = _mm(x, W) * norm_s[:, None]
        aggp = _segsum_kernel(y, g4)
        agg = aggp[0, :N] + aggp[1, :N]
        return agg * norm_d[:, None] + b

    h_n = jax.nn.relu(gconv(x_n, Wn1, bn1))
    h_n = gconv(h_n, Wn2, bn2)

    g2 = g.reshape(2, NCHS, SCK)
    lists, cnts = _segmax_kernel(g2)

    def seg_max(a):
        ap = _segmax_apply(a, lists, cnts)
        return ap[:, :NSEG, :].reshape(NW * NSEG, 64)[:N]

    a1 = _mm(x_e, Th1)
    c1 = _mm(x_e, Ph1 - Th1) + Tb1 + Pb1
    m1 = seg_max(a1)
    h_e = jax.nn.relu(jnp.where(deg_in[:, None] > 0, m1 + c1, 0.0))

    a2 = _mm(h_e, Th2)
    c2 = _mm(h_e, Ph2 - Th2) + Tb2 + Pb2
    m2 = seg_max(a2)
    h_e = jnp.where(deg_in[:, None] > 0, m2 + c2, 0.0)

    h = jnp.concatenate([h_n, h_e], axis=1)
    score = _score_kernel(h, g4)
    return score.reshape(E, 1)


# speedup vs baseline: 1.0184x; 1.0184x over previous
import functools
import jax
import jax.numpy as jnp
from jax import lax
from jax.experimental import pallas as pl
from jax.experimental.pallas import tpu as pltpu
from jax.experimental.pallas import tpu_sc as plsc

N = 10000
E = 320000
NC, NS, L = 2, 16, 16
NW = NC * NS
EPW = E // NW
CK = 80
NCH = EPW // CK
NPAD = 10240


def _zero_vmem(buf, n):
    zeros = jnp.zeros((L,), jnp.float32)
    for i in range(n // L):
        buf[pl.ds(i * L, L)] = zeros


_MESH = plsc.VectorSubcoreMesh(core_axis_name="c", subcore_axis_name="s")


@functools.partial(
    pl.kernel,
    out_type=jax.ShapeDtypeStruct((NC, 2, NPAD), jnp.float32),
    mesh=_MESH,
    scratch_types=[
        pltpu.VMEM((NCH, CK), jnp.int32),
        pltpu.VMEM((NCH, CK), jnp.int32),
        pltpu.VMEM((CK,), jnp.float32),
        pltpu.VMEM((640,), jnp.float32),
        pltpu.VMEM_SHARED((NPAD,), jnp.float32),
        pltpu.VMEM_SHARED((NPAD,), jnp.float32),
        pltpu.SemaphoreType.DMA,
    ],
)
def _deg_kernel(g_hbm, out_hbm, sbuf, dbuf, ones, zbuf, acc_s, acc_d, dsem):
    c = lax.axis_index("c")
    s = lax.axis_index("s")
    one = jnp.ones((L,), jnp.float32)
    for i in range(CK // L):
        ones[pl.ds(i * L, L)] = one
    _zero_vmem(zbuf, 640)
    pltpu.sync_copy(zbuf, acc_s.at[pl.ds(s * 640, 640)])
    pltpu.sync_copy(zbuf, acc_d.at[pl.ds(s * 640, 640)])
    plsc.subcore_barrier()
    wid = s * NC + c
    pltpu.sync_copy(g_hbm.at[0, wid], sbuf)
    pltpu.sync_copy(g_hbm.at[1, wid], dbuf)

    def body(j, _):
        pltpu.async_copy(ones, acc_s.at[sbuf.at[j]], dsem, add=True)
        pltpu.async_copy(ones, acc_d.at[dbuf.at[j]], dsem, add=True)

        @pl.when(j >= 4)
        def _():
            pltpu.make_async_copy(ones, acc_s.at[sbuf.at[j - 4]], dsem).wait()
            pltpu.make_async_copy(ones, acc_d.at[dbuf.at[j - 4]], dsem).wait()

        return _

    lax.fori_loop(0, NCH, body, None)

    def drain(j, _):
        pltpu.make_async_copy(ones, acc_s.at[sbuf.at[j]], dsem).wait()
        pltpu.make_async_copy(ones, acc_d.at[dbuf.at[j]], dsem).wait()
        return _

    lax.fori_loop(NCH - 4, NCH, drain, None)
    plsc.subcore_barrier()

    @pl.when(s == 0)
    def _():
        pltpu.sync_copy(acc_s, out_hbm.at[c, 0])
        pltpu.sync_copy(acc_d, out_hbm.at[c, 1])


@functools.partial(
    pl.kernel,
    out_type=jax.ShapeDtypeStruct((NC, NPAD, 64), jnp.float32),
    mesh=_MESH,
    scratch_types=[
        pltpu.VMEM((NCH, CK), jnp.int32),
        pltpu.VMEM((NCH, CK), jnp.int32),
        pltpu.VMEM((CK, 64), jnp.float32),
        pltpu.VMEM((CK, 64), jnp.float32),
        pltpu.VMEM((64, 64), jnp.float32),
        pltpu.VMEM_SHARED((NPAD, 64), jnp.float32),
        pltpu.SemaphoreType.DMA,
        pltpu.SemaphoreType.DMA,
        pltpu.SemaphoreType.DMA,
        pltpu.SemaphoreType.DMA,
    ],
    compiler_params=pltpu.CompilerParams(use_tc_tiling_on_sc=False),
)
def _segsum_kernel(tbl_hbm, g_hbm, out_hbm, sbuf, dbuf, rb0, rb1, zrows, acc,
                   sem0, sem1, ssem0, ssem1):
    c = lax.axis_index("c")
    s = lax.axis_index("s")
    zeros = jnp.zeros((L,), jnp.float32)

    def zrow_body(i, _):
        for q in range(4):
            zrows[i, pl.ds(q * L, L)] = zeros
        return _

    lax.fori_loop(0, 64, zrow_body, None)

    def zacc_body(k, _):
        pltpu.sync_copy(zrows, acc.at[pl.ds(s * 640 + k * 64, 64)])
        return _

    lax.fori_loop(0, 10, zacc_body, None)
    plsc.subcore_barrier()

    wid = s * NC + c
    pltpu.sync_copy(g_hbm.at[0, wid], sbuf)
    pltpu.sync_copy(g_hbm.at[1, wid], dbuf)

    rbs = (rb0, rb1)
    sems = (sem0, sem1)
    ssems = (ssem0, ssem1)

    def wait_scatter(b, j):
        pltpu.make_async_copy(rbs[b], acc.at[dbuf.at[j]], ssems[b]).wait()

    pltpu.async_copy(tbl_hbm.at[sbuf.at[0]], rb0, sem0)

    def body(jj, _):
        for b in range(2):
            j = jj * 2 + b

            @pl.when(j >= 1)
            def _():
                wait_scatter(1 - b, j - 1)

            pltpu.async_copy(tbl_hbm.at[sbuf.at[j + 1]], rbs[1 - b], sems[1 - b])
            pltpu.make_async_copy(tbl_hbm.at[sbuf.at[j]], rbs[b], sems[b]).wait()
            pltpu.async_copy(rbs[b], acc.at[dbuf.at[j]], ssems[b], add=True)
        return _

    lax.fori_loop(0, (NCH - 1) // 2, body, None)
    j_last = NCH - 1
    wait_scatter(1, j_last - 1)
    pltpu.make_async_copy(tbl_hbm.at[sbuf.at[j_last]], rb0, sem0).wait()
    pltpu.async_copy(rb0, acc.at[dbuf.at[j_last]], ssem0, add=True)
    wait_scatter(0, j_last)
    plsc.subcore_barrier()
    pltpu.sync_copy(acc.at[pl.ds(s * 640, 640)], out_hbm.at[c, pl.ds(s * 640, 640)])


@functools.partial(
    pl.kernel,
    out_type=jax.ShapeDtypeStruct((E,), jnp.float32),
    mesh=_MESH,
    scratch_types=[
        pltpu.VMEM((NCH, CK), jnp.int32),
        pltpu.VMEM((NCH, CK), jnp.int32),
        pltpu.VMEM((CK, 128), jnp.float32),
        pltpu.VMEM((CK, 128), jnp.float32),
        pltpu.VMEM((CK, 128), jnp.float32),
        pltpu.VMEM((CK, 128), jnp.float32),
        pltpu.VMEM((CK,), jnp.float32),
        pltpu.VMEM((CK,), jnp.float32),
        pltpu.SemaphoreType.DMA,
        pltpu.SemaphoreType.DMA,
        pltpu.SemaphoreType.DMA,
        pltpu.SemaphoreType.DMA,
    ],
    compiler_params=pltpu.CompilerParams(
        use_tc_tiling_on_sc=False, needs_layout_passes=False
    ),
)
def _score_kernel(h_hbm, g_hbm, out_hbm, sbuf, dbuf, rs0, rd0, rs1, rd1, obuf0,
                  obuf1, sem0, sem1, osem0, osem1):
    c = lax.axis_index("c")
    s = lax.axis_index("s")
    wid = s * NC + c
    pltpu.sync_copy(g_hbm.at[0, wid], sbuf)
    pltpu.sync_copy(g_hbm.at[1, wid], dbuf)

    rss = (rs0, rs1)
    rds = (rd0, rd1)
    sems = (sem0, sem1)
    obufs = (obuf0, obuf1)
    osems = (osem0, osem1)

    def fire(j, b):
        pltpu.async_copy(h_hbm.at[sbuf.at[j]], rss[b], sems[b])
        pltpu.async_copy(h_hbm.at[dbuf.at[j]], rds[b], sems[b])

    def wait(j, b):
        pltpu.make_async_copy(h_hbm.at[sbuf.at[j]], rss[b], sems[b]).wait()
        pltpu.make_async_copy(h_hbm.at[dbuf.at[j]], rds[b], sems[b]).wait()

    def process(j, b):
        rs, rd = rss[b], rds[b]
        obuf = obufs[b]

        @pl.when(j >= 2)
        def _():
            pltpu.make_async_copy(
                obuf, out_hbm.at[pl.ds(wid * EPW + (j - 2) * CK, CK)], osems[b]
            ).wait()

        lanes = lax.iota(jnp.int32, L)

        def grp_body(gi, _):
            res = jnp.zeros((L,), jnp.float32)
            for t in range(L):
                e = gi * L + t
                p = [rs[e, pl.ds(q * L, L)] * rd[e, pl.ds(q * L, L)]
                     for q in range(8)]
                s1 = [p[0] + p[1], p[2] + p[3], p[4] + p[5], p[6] + p[7]]
                acc = (s1[0] + s1[1]) + (s1[2] + s1[3])
                res = jnp.where(lanes == t, jnp.sum(acc), res)
            obuf[pl.ds(gi * L, L)] = res
            return _

        lax.fori_loop(0, CK // L, grp_body, None)
        pltpu.async_copy(obuf, out_hbm.at[pl.ds(wid * EPW + j * CK, CK)], osems[b])

    fire(0, 0)

    def body(jj, _):
        for b in range(2):
            j = jj * 2 + b
            wait(j, b)
            fire(j + 1, 1 - b)
            process(j, b)
        return _

    lax.fori_loop(0, (NCH - 1) // 2, body, None)
    j_last = NCH - 1
    wait(j_last, 0)
    process(j_last, 0)
    pltpu.make_async_copy(
        obuf0, out_hbm.at[pl.ds(wid * EPW + j_last * CK, CK)], osem0).wait()
    pltpu.make_async_copy(
        obuf1, out_hbm.at[pl.ds(wid * EPW + (j_last - 1) * CK, CK)], osem1).wait()


NSEG = 313
ACC_R = 320
SCK = 2560
NCHS = E // SCK
NEG = -1e30
CAPB = 128
CAP = CAPB * 128


@functools.partial(
    pl.kernel,
    out_type=(
        jax.ShapeDtypeStruct((2, NW, CAP), jnp.int32),
        jax.ShapeDtypeStruct((NW, L), jnp.int32),
    ),
    mesh=_MESH,
    scratch_types=[
        pltpu.VMEM((SCK,), jnp.int32),
        pltpu.VMEM((SCK,), jnp.int32),
        pltpu.VMEM((SCK,), jnp.int32),
        pltpu.VMEM((SCK,), jnp.int32),
        pltpu.VMEM((256,), jnp.int32),
        pltpu.VMEM((256,), jnp.int32),
        pltpu.VMEM((L,), jnp.int32),
        pltpu.VMEM((128,), jnp.int32),
        pltpu.VMEM((128,), jnp.int32),
        pltpu.SemaphoreType.DMA,
        pltpu.SemaphoreType.DMA,
        pltpu.SemaphoreType.DMA,
    ],
    compiler_params=pltpu.CompilerParams(
        use_tc_tiling_on_sc=False, needs_layout_passes=False
    ),
)
def _segmax_kernel(g_hbm, lists_hbm, cnt_hbm, sb0, db0, sb1,
                   db1, psrc, pdst, cbuf, flsrc, fldst, sem0, sem1, lsem):
    c = lax.axis_index("c")
    s = lax.axis_index("s")
    wid = s * NC + c
    lo = wid * NSEG

    sbs = (sb0, sb1)
    dbs = (db0, db1)
    sems = (sem0, sem1)

    def fire(j, b):
        pltpu.async_copy(g_hbm.at[0, j], sbs[b], sems[b])
        pltpu.async_copy(g_hbm.at[1, j], dbs[b], sems[b])

    def wait(j, b):
        pltpu.make_async_copy(g_hbm.at[0, j], sbs[b], sems[b]).wait()
        pltpu.make_async_copy(g_hbm.at[1, j], dbs[b], sems[b]).wait()

    def wait_listwrite(nf_prev):
        @pl.when((nf_prev >= 0) & (nf_prev < CAPB))
        def _():
            pltpu.make_async_copy(
                flsrc, lists_hbm.at[0, wid, pl.ds(nf_prev * 128, 128)], lsem
            ).wait()
            pltpu.make_async_copy(
                fldst, lists_hbm.at[1, wid, pl.ds(nf_prev * 128, 128)], lsem
            ).wait()

    def flush(st):
        np, nf = st
        wait_listwrite(nf - 1)
        for i in range(8):
            flsrc[pl.ds(i * L, L)] = psrc[pl.ds(i * L, L)]
            fldst[pl.ds(i * L, L)] = pdst[pl.ds(i * L, L)]

        @pl.when(nf < CAPB)
        def _():
            pltpu.async_copy(flsrc, lists_hbm.at[0, wid, pl.ds(nf * 128, 128)],
                             lsem)
            pltpu.async_copy(fldst, lists_hbm.at[1, wid, pl.ds(nf * 128, 128)],
                             lsem)

        for i in range(8):
            psrc[pl.ds(i * L, L)] = psrc[pl.ds(128 + i * L, L)]
            pdst[pl.ds(i * L, L)] = pdst[pl.ds(128 + i * L, L)]
        return (np - 128, nf + 1)

    def maybe_flush(st):
        return lax.cond(st[0] >= 128, flush, lambda x: x, st)

    def process(b, st):
        sb, db = sbs[b], dbs[b]

        def batch(k, st):
            np, nf = st
            for i8 in range(8):
                base = (k * 8 + i8) * L
                sV = sb[pl.ds(base, L)]
                dV = db[pl.ds(base, L)]
                dl = dV - lo
                m = (dl >= 0) & (dl < NSEG)
                cnt = plsc.all_reduce_population_count(m)[0]
                plsc.store_compressed(psrc.at[pl.ds(np, L)], sV, mask=m)
                plsc.store_compressed(pdst.at[pl.ds(np, L)], dl, mask=m)
                np = np + cnt
            return maybe_flush((np, nf))

        return lax.fori_loop(0, SCK // 128, batch, st)

    fire(0, 0)

    def chunk_body(jj, st):
        for b in range(2):
            j = jj * 2 + b
            wait(j, b)
            fire(j + 1, 1 - b)
            st = process(b, st)
        return st

    st = lax.fori_loop(0, (NCHS - 1) // 2, chunk_body, (0, 0))
    j_last = NCHS - 1
    wait(j_last, 0)
    st = process(0, st)
    st = maybe_flush(st)

    zero16 = jnp.zeros((L,), jnp.int32)
    trash16 = jnp.full((L,), NSEG, jnp.int32)
    full = jnp.ones((L,), jnp.bool_)

    def pad16(st):
        np, nf = st
        plsc.store_compressed(psrc.at[pl.ds(np, L)], zero16, mask=full)
        plsc.store_compressed(pdst.at[pl.ds(np, L)], trash16, mask=full)
        return (np + L, nf)

    def pad_body(_, st):
        return lax.cond(st[0] < 128, pad16, lambda x: x, st)

    st = lax.fori_loop(0, 8, pad_body, st)
    st = flush(st)
    nf = st[1]
    wait_listwrite(nf - 1)

    for i in range(8):
        psrc[pl.ds(i * L, L)] = zero16
        pdst[pl.ds(i * L, L)] = trash16

    def padblk(nf):
        @pl.when(nf < CAPB)
        def _():
            pltpu.sync_copy(psrc.at[pl.ds(0, 128)],
                            lists_hbm.at[0, wid, pl.ds(nf * 128, 128)])
            pltpu.sync_copy(pdst.at[pl.ds(0, 128)],
                            lists_hbm.at[1, wid, pl.ds(nf * 128, 128)])
        return nf + 1

    def padblk_body(_, nf):
        return lax.cond(nf % 8 != 0, padblk, lambda x: x, nf)

    nf = lax.fori_loop(0, 7, padblk_body, nf)
    cbuf[pl.ds(0, L)] = jnp.full((L,), 0, jnp.int32) + (nf // 8)
    pltpu.sync_copy(cbuf, cnt_hbm.at[wid])


@functools.partial(
    pl.kernel,
    out_type=jax.ShapeDtypeStruct((NW, ACC_R, 64), jnp.float32),
    mesh=_MESH,
    scratch_types=[
        pltpu.VMEM((1024,), jnp.int32),
        pltpu.VMEM((1024,), jnp.int32),
        pltpu.VMEM((128, 64), jnp.float32),
        pltpu.VMEM((128, 64), jnp.float32),
        pltpu.VMEM((ACC_R, 64), jnp.float32),
        pltpu.VMEM((L,), jnp.int32),
        pltpu.SemaphoreType.DMA,
        pltpu.SemaphoreType.DMA,
    ],
    compiler_params=pltpu.CompilerParams(
        use_tc_tiling_on_sc=False, needs_layout_passes=False
    ),
)
def _segmax_apply(tbl_hbm, lists_hbm, cnt_hbm, out_hbm, lsrc, ldst, rf0, rf1,
                  accr, cbuf, sem0, sem1):
    c = lax.axis_index("c")
    s = lax.axis_index("s")
    wid = s * NC + c

    neg = jnp.full((L,), NEG, jnp.float32)

    def init_body(rr, _):
        for q in range(4):
            accr[rr, pl.ds(q * L, L)] = neg
        return _

    lax.fori_loop(0, ACC_R, init_body, None)

    pltpu.sync_copy(cnt_hbm.at[wid], cbuf)
    nch = cbuf[pl.ds(0, L)][0]

    rfs = (rf0, rf1)
    sems = (sem0, sem1)

    def chunk_body(cc, _):
        pltpu.sync_copy(lists_hbm.at[0, wid, pl.ds(cc * 1024, 1024)], lsrc)
        pltpu.sync_copy(lists_hbm.at[1, wid, pl.ds(cc * 1024, 1024)], ldst)
        pltpu.async_copy(tbl_hbm.at[lsrc.at[pl.ds(0, 128)]], rf0, sem0)
        for b in range(8):
            pb = b % 2
            pltpu.make_async_copy(
                tbl_hbm.at[lsrc.at[pl.ds(b * 128, 128)]], rfs[pb], sems[pb]
            ).wait()
            if b < 7:
                pltpu.async_copy(
                    tbl_hbm.at[lsrc.at[pl.ds((b + 1) * 128, 128)]],
                    rfs[1 - pb], sems[1 - pb])
            rf = rfs[pb]

            def fold(gg, _):
                dv = ldst[pl.ds(b * 128 + gg * L, L)]
                for tt in range(L // 2):
                    e0 = gg * L + 2 * tt
                    e1 = e0 + 1
                    d0 = dv[2 * tt]
                    d1 = dv[2 * tt + 1]
                    cur0 = [accr[d0, pl.ds(q * L, L)] for q in range(4)]
                    cur1 = [accr[d1, pl.ds(q * L, L)] for q in range(4)]
                    new0 = [rf[e0, pl.ds(q * L, L)] for q in range(4)]
                    new1 = [rf[e1, pl.ds(q * L, L)] for q in range(4)]
                    same = d0 == d1
                    for q in range(4):
                        n1 = jnp.where(same, jnp.maximum(new0[q], new1[q]),
                                       new1[q])
                        accr[d0, pl.ds(q * L, L)] = jnp.maximum(cur0[q], new0[q])
                        accr[d1, pl.ds(q * L, L)] = jnp.maximum(cur1[q], n1)
                return _

            lax.fori_loop(0, 128 // L, fold, None)
        return _

    lax.fori_loop(0, nch, chunk_body, None)
    pltpu.sync_copy(accr, out_hbm.at[wid])


def _mm_body(x_ref, w_ref, o_ref):
    o_ref[...] = jnp.dot(x_ref[...], w_ref[...],
                         preferred_element_type=jnp.float32)


def _mm(x, w):
    M, K = x.shape
    F = w.shape[1]
    return pl.pallas_call(
        _mm_body,
        grid=(10,),
        in_specs=[
            pl.BlockSpec((M // 10, K), lambda i: (i, 0)),
            pl.BlockSpec((K, F), lambda i: (0, 0)),
        ],
        out_specs=pl.BlockSpec((M // 10, F), lambda i: (i, 0)),
        out_shape=jax.ShapeDtypeStruct((M, F), jnp.float32),
    )(x, w)


def kernel(g, x_n, x_e, Wn1, bn1, Wn2, bn2, Th1, Tb1, Ph1, Pb1, Th2, Tb2, Ph2, Pb2):
    src, dst = g[0], g[1]
    g4 = g.reshape(2, NW, NCH, CK)

    degp = _deg_kernel(g4)
    deg_out = degp[0, 0, :N] + degp[1, 0, :N]
    deg_in = degp[0, 1, :N] + degp[1, 1, :N]
    norm_s = jnp.where(deg_out > 0, deg_out, 1.0) ** -0.5
    norm_d = jnp.where(deg_in > 0, deg_in, 1.0) ** -0.5

    def gconv(x, W, b):
        y = _mm(x, W) * norm_s[:, None]
        aggp = _segsum_kernel(y, g4)
        agg = aggp[0, :N] + aggp[1, :N]
        return agg * norm_d[:, None] + b

    h_n = jax.nn.relu(gconv(x_n, Wn1, bn1))
    h_n = gconv(h_n, Wn2, bn2)

    g2 = g.reshape(2, NCHS, SCK)
    lists, cnts = _segmax_kernel(g2)

    def seg_max(a):
        ap = _segmax_apply(a, lists, cnts)
        return ap[:, :NSEG, :].reshape(NW * NSEG, 64)[:N]

    a1 = _mm(x_e, Th1)
    c1 = _mm(x_e, Ph1 - Th1) + Tb1 + Pb1
    m1 = seg_max(a1)
    h_e = jax.nn.relu(jnp.where(deg_in[:, None] > 0, m1 + c1, 0.0))

    a2 = _mm(h_e, Th2)
    c2 = _mm(h_e, Ph2 - Th2) + Tb2 + Pb2
    m2 = seg_max(a2)
    h_e = jnp.where(deg_in[:, None] > 0, m2 + c2, 0.0)

    h = jnp.concatenate([h_n, h_e], axis=1)
    score = _score_kernel(h, g4)
    return score.reshape(E, 1)
